# baseline (speedup 1.0000x reference)
import functools

import jax
import jax.numpy as jnp
from jax import lax
from jax.experimental import pallas as pl
from jax.experimental.pallas import tpu as pltpu
from jax.experimental.pallas import tpu_sc as plsc

FEATURES = 32
NUM_CORES = 2
NUM_SUBCORES = 16
NUM_WORKERS = NUM_CORES * NUM_SUBCORES

K = 128
NK = 16
CHUNK = K * NK


def _embed_body(n_chunks, idx_hbm, table_hbm, out_hbm, idx_v, rows_v, sem):
    wid = lax.axis_index("s") * NUM_CORES + lax.axis_index("c")
    idx_rows_per_w = n_chunks * NK

    def chunk_body(g, carry):
        idx_row0 = wid * idx_rows_per_w + g * NK
        out_row0 = idx_row0 * K
        pltpu.sync_copy(idx_hbm.at[pl.ds(idx_row0, NK)], idx_v)
        copies = []
        for j in range(NK):
            copies.append(
                pltpu.async_copy(
                    table_hbm.at[idx_v.at[j]],
                    rows_v.at[pl.ds(j * K, K)],
                    sem,
                )
            )
        for c in copies:
            c.wait()
        pltpu.sync_copy(rows_v, out_hbm.at[pl.ds(out_row0, CHUNK)])
        return carry

    lax.fori_loop(0, n_chunks, chunk_body, 0)


def kernel(inputs, embedding):
    batch, hist = inputs.shape
    total = batch * hist
    assert total % (NUM_WORKERS * CHUNK) == 0
    n_chunks = total // (NUM_WORKERS * CHUNK)

    idx2d = inputs.reshape(total // K, K)

    num_emb = embedding.shape[0]
    feat32 = FEATURES // 2
    flat = jax.lax.bitcast_convert_type(embedding, jnp.uint16).reshape(-1)
    lo = flat[0::2].astype(jnp.uint32)
    hi = flat[1::2].astype(jnp.uint32)
    table_i32 = jax.lax.bitcast_convert_type(
        lo | (hi << 16), jnp.int32
    ).reshape(num_emb, feat32)

    mesh = plsc.VectorSubcoreMesh(core_axis_name="c", subcore_axis_name="s")
    run = pl.kernel(
        functools.partial(_embed_body, n_chunks),
        out_type=jax.ShapeDtypeStruct((total, feat32), jnp.int32),
        mesh=mesh,
        scratch_types=[
            pltpu.VMEM((NK, K), jnp.int32),
            pltpu.VMEM((CHUNK, feat32), jnp.int32),
            pltpu.SemaphoreType.DMA,
        ],
        compiler_params=pltpu.CompilerParams(use_tc_tiling_on_sc=False),
    )
    out = run(idx2d, table_i32)
    out_bf16 = jax.lax.bitcast_convert_type(out, jnp.bfloat16)
    return out_bf16.reshape(batch, hist, FEATURES)


# speedup vs baseline: 4.6571x; 4.6571x over previous
import functools

import jax
import jax.numpy as jnp
from jax import lax
from jax.experimental import pallas as pl
from jax.experimental.pallas import tpu as pltpu
from jax.experimental.pallas import tpu_sc as plsc

FEATURES = 32
NUM_CORES = 2
NUM_SUBCORES = 16
NUM_WORKERS = NUM_CORES * NUM_SUBCORES

K = 128
NK = 16
CHUNK = K * NK

PACK_CHUNK = 3125


def _pack_body(n_chunks, u16_hbm, out_hbm, buf, obuf, _sem):
    wid = lax.axis_index("s") * NUM_CORES + lax.axis_index("c")
    rows_per_w = n_chunks * PACK_CHUNK
    unroll = 5

    def chunk_body(g, carry):
        row0 = wid * rows_per_w + g * PACK_CHUNK
        pltpu.sync_copy(u16_hbm.at[pl.ds(row0, PACK_CHUNK)], buf)

        def repack(i, c):
            for u in range(unroll):
                r = i * unroll + u
                obuf[r, :] = plsc.bitcast(buf[r, :], jnp.int32)
            return c

        lax.fori_loop(0, PACK_CHUNK // unroll, repack, 0)
        pltpu.sync_copy(obuf, out_hbm.at[pl.ds(row0, PACK_CHUNK)])
        return carry

    lax.fori_loop(0, n_chunks, chunk_body, 0)


def _embed_body(n_chunks, idx_hbm, table_hbm, out_hbm, idx_v, rows_v, sem):
    wid = lax.axis_index("s") * NUM_CORES + lax.axis_index("c")
    idx_rows_per_w = n_chunks * NK

    def chunk_body(g, carry):
        idx_row0 = wid * idx_rows_per_w + g * NK
        out_row0 = idx_row0 * K
        pltpu.sync_copy(idx_hbm.at[pl.ds(idx_row0, NK)], idx_v)
        copies = []
        for j in range(NK):
            copies.append(
                pltpu.async_copy(
                    table_hbm.at[idx_v.at[j]],
                    rows_v.at[pl.ds(j * K, K)],
                    sem,
                )
            )
        for c in copies:
            c.wait()
        pltpu.sync_copy(rows_v, out_hbm.at[pl.ds(out_row0, CHUNK)])
        return carry

    lax.fori_loop(0, n_chunks, chunk_body, 0)


def kernel(inputs, embedding):
    batch, hist = inputs.shape
    total = batch * hist
    assert total % (NUM_WORKERS * CHUNK) == 0
    n_chunks = total // (NUM_WORKERS * CHUNK)

    num_emb = embedding.shape[0]
    feat32 = FEATURES // 2
    assert num_emb % (NUM_WORKERS * PACK_CHUNK) == 0
    n_pack_chunks = num_emb // (NUM_WORKERS * PACK_CHUNK)

    idx2d = inputs.reshape(total // K, K)
    table_u16 = jax.lax.bitcast_convert_type(embedding, jnp.uint16)

    mesh = plsc.VectorSubcoreMesh(core_axis_name="c", subcore_axis_name="s")
    sc_params = pltpu.CompilerParams(
        use_tc_tiling_on_sc=False, needs_layout_passes=False
    )

    pack = pl.kernel(
        functools.partial(_pack_body, n_pack_chunks),
        out_type=jax.ShapeDtypeStruct((num_emb, feat32), jnp.int32),
        mesh=mesh,
        scratch_types=[
            pltpu.VMEM((PACK_CHUNK, FEATURES), jnp.uint16),
            pltpu.VMEM((PACK_CHUNK, feat32), jnp.int32),
            pltpu.SemaphoreType.DMA,
        ],
        compiler_params=sc_params,
    )
    table_i32 = pack(table_u16)

    run = pl.kernel(
        functools.partial(_embed_body, n_chunks),
        out_type=jax.ShapeDtypeStruct((total, feat32), jnp.int32),
        mesh=mesh,
        scratch_types=[
            pltpu.VMEM((NK, K), jnp.int32),
            pltpu.VMEM((CHUNK, feat32), jnp.int32),
            pltpu.SemaphoreType.DMA,
        ],
        compiler_params=sc_params,
    )
    out = run(idx2d, table_i32)
    out_bf16 = jax.lax.bitcast_convert_type(out, jnp.bfloat16)
    return out_bf16.reshape(batch, hist, FEATURES)


# speedup vs baseline: 6.1214x; 1.3144x over previous
import functools

import jax
import jax.numpy as jnp
from jax import lax
from jax.experimental import pallas as pl
from jax.experimental.pallas import tpu as pltpu
from jax.experimental.pallas import tpu_sc as plsc

FEATURES = 32
NUM_CORES = 2
NUM_SUBCORES = 16
NUM_WORKERS = NUM_CORES * NUM_SUBCORES

B_CHUNK = 128
FIRE = 16

PACK_CHUNK = 3125


def _pack_body(n_chunks, u16_hbm, out_hbm, buf, obuf, _sem):
    wid = lax.axis_index("s") * NUM_CORES + lax.axis_index("c")
    rows_per_w = n_chunks * PACK_CHUNK
    unroll = 5

    def chunk_body(g, carry):
        row0 = wid * rows_per_w + g * PACK_CHUNK
        pltpu.sync_copy(u16_hbm.at[pl.ds(row0, PACK_CHUNK)], buf)

        def repack(i, c):
            for u in range(unroll):
                r = i * unroll + u
                obuf[r, :] = plsc.bitcast(buf[r, :], jnp.int32)
            return c

        lax.fori_loop(0, PACK_CHUNK // unroll, repack, 0)
        pltpu.sync_copy(obuf, out_hbm.at[pl.ds(row0, PACK_CHUNK)])
        return carry

    lax.fori_loop(0, n_chunks, chunk_body, 0)


def _embed_body(n_chunks, hist, idx_hbm, table_hbm, out_hbm, idx_v, rows_v, sem):
    wid = lax.axis_index("s") * NUM_CORES + lax.axis_index("c")
    b_per_w = n_chunks * B_CHUNK

    def chunk_body(g, carry):
        b0 = wid * b_per_w + g * B_CHUNK
        pltpu.sync_copy(idx_hbm.at[pl.ds(b0, B_CHUNK)], idx_v)

        def fire_batch(ro, c):
            copies = []
            for u in range(FIRE):
                r = ro * FIRE + u
                copies.append(
                    pltpu.async_copy(
                        table_hbm.at[idx_v.at[r]],
                        rows_v.at[r],
                        sem,
                    )
                )
            for cp in copies:
                cp.wait()
            return c

        lax.fori_loop(0, B_CHUNK // FIRE, fire_batch, 0)
        pltpu.sync_copy(rows_v, out_hbm.at[pl.ds(b0, B_CHUNK)])
        return carry

    lax.fori_loop(0, n_chunks, chunk_body, 0)


def kernel(inputs, embedding):
    batch, hist = inputs.shape
    assert batch % (NUM_WORKERS * B_CHUNK) == 0
    n_chunks = batch // (NUM_WORKERS * B_CHUNK)

    num_emb = embedding.shape[0]
    feat32 = FEATURES // 2
    assert num_emb % (NUM_WORKERS * PACK_CHUNK) == 0
    n_pack_chunks = num_emb // (NUM_WORKERS * PACK_CHUNK)

    table_u16 = jax.lax.bitcast_convert_type(embedding, jnp.uint16)

    mesh = plsc.VectorSubcoreMesh(core_axis_name="c", subcore_axis_name="s")
    sc_params = pltpu.CompilerParams(
        use_tc_tiling_on_sc=False, needs_layout_passes=False
    )

    pack = pl.kernel(
        functools.partial(_pack_body, n_pack_chunks),
        out_type=jax.ShapeDtypeStruct((num_emb, feat32), jnp.int32),
        mesh=mesh,
        scratch_types=[
            pltpu.VMEM((PACK_CHUNK, FEATURES), jnp.uint16),
            pltpu.VMEM((PACK_CHUNK, feat32), jnp.int32),
            pltpu.SemaphoreType.DMA,
        ],
        compiler_params=sc_params,
    )
    table_i32 = pack(table_u16)

    run = pl.kernel(
        functools.partial(_embed_body, n_chunks, hist),
        out_type=jax.ShapeDtypeStruct((batch, hist, feat32), jnp.int32),
        mesh=mesh,
        scratch_types=[
            pltpu.VMEM((B_CHUNK, hist), jnp.int32),
            pltpu.VMEM((B_CHUNK, hist, feat32), jnp.int32),
            pltpu.SemaphoreType.DMA,
        ],
        compiler_params=sc_params,
    )
    out = run(inputs, table_i32)
    out_bf16 = jax.lax.bitcast_convert_type(out, jnp.bfloat16)
    return out_bf16.reshape(batch, hist, FEATURES)


# speedup vs baseline: 6.8397x; 1.1173x over previous
import functools

import jax
import jax.numpy as jnp
from jax import lax
from jax.experimental import pallas as pl
from jax.experimental.pallas import tpu as pltpu
from jax.experimental.pallas import tpu_sc as plsc

FEATURES = 32
NUM_CORES = 2
NUM_SUBCORES = 16
NUM_WORKERS = NUM_CORES * NUM_SUBCORES

B_CHUNK = 128
FIRE = 16

PACK_CHUNK = 3125


def _pack_body(n_chunks, u16_hbm, out_hbm, buf, obuf, _sem):
    wid = lax.axis_index("s") * NUM_CORES + lax.axis_index("c")
    rows_per_w = n_chunks * PACK_CHUNK
    unroll = 5

    def chunk_body(g, carry):
        row0 = wid * rows_per_w + g * PACK_CHUNK
        pltpu.sync_copy(u16_hbm.at[pl.ds(row0, PACK_CHUNK)], buf)

        def repack(i, c):
            for u in range(unroll):
                r = i * unroll + u
                obuf[r, :] = plsc.bitcast(buf[r, :], jnp.int32)
            return c

        lax.fori_loop(0, PACK_CHUNK // unroll, repack, 0)
        pltpu.sync_copy(obuf, out_hbm.at[pl.ds(row0, PACK_CHUNK)])
        return carry

    lax.fori_loop(0, n_chunks, chunk_body, 0)


def _embed_body(n_chunks, hist, idx_hbm, table_hbm, out_hbm, idx_v, rows_v, obuf, sem):
    wid = lax.axis_index("s") * NUM_CORES + lax.axis_index("c")
    b_per_w = n_chunks * B_CHUNK
    feat32 = FEATURES // 2

    def chunk_body(g, carry):
        b0 = wid * b_per_w + g * B_CHUNK
        pltpu.sync_copy(idx_hbm.at[pl.ds(b0, B_CHUNK)], idx_v)

        def fire_batch(ro, c):
            copies = []
            for u in range(FIRE):
                r = ro * FIRE + u
                copies.append(
                    pltpu.async_copy(
                        table_hbm.at[idx_v.at[r]],
                        rows_v.at[r],
                        sem,
                    )
                )
            for cp in copies:
                cp.wait()
            return c

        lax.fori_loop(0, B_CHUNK // FIRE, fire_batch, 0)

        def to_u16(i, c):
            for u in range(4):
                r = i * 4 + u
                b = r // hist
                h = r % hist
                obuf[b, h, :] = plsc.bitcast(rows_v[b, h, :], jnp.uint16)
            return c

        lax.fori_loop(0, B_CHUNK * hist // 4, to_u16, 0)
        pltpu.sync_copy(obuf, out_hbm.at[pl.ds(b0, B_CHUNK)])
        return carry

    lax.fori_loop(0, n_chunks, chunk_body, 0)


def kernel(inputs, embedding):
    batch, hist = inputs.shape
    assert batch % (NUM_WORKERS * B_CHUNK) == 0
    n_chunks = batch // (NUM_WORKERS * B_CHUNK)

    num_emb = embedding.shape[0]
    feat32 = FEATURES // 2
    assert num_emb % (NUM_WORKERS * PACK_CHUNK) == 0
    n_pack_chunks = num_emb // (NUM_WORKERS * PACK_CHUNK)

    table_u16 = jax.lax.bitcast_convert_type(embedding, jnp.uint16)

    mesh = plsc.VectorSubcoreMesh(core_axis_name="c", subcore_axis_name="s")
    sc_params = pltpu.CompilerParams(
        use_tc_tiling_on_sc=False, needs_layout_passes=False
    )

    pack = pl.kernel(
        functools.partial(_pack_body, n_pack_chunks),
        out_type=jax.ShapeDtypeStruct((num_emb, feat32), jnp.int32),
        mesh=mesh,
        scratch_types=[
            pltpu.VMEM((PACK_CHUNK, FEATURES), jnp.uint16),
            pltpu.VMEM((PACK_CHUNK, feat32), jnp.int32),
            pltpu.SemaphoreType.DMA,
        ],
        compiler_params=sc_params,
    )
    table_i32 = pack(table_u16)

    run = pl.kernel(
        functools.partial(_embed_body, n_chunks, hist),
        out_type=jax.ShapeDtypeStruct((batch, hist, FEATURES), jnp.uint16),
        mesh=mesh,
        scratch_types=[
            pltpu.VMEM((B_CHUNK, hist), jnp.int32),
            pltpu.VMEM((B_CHUNK, hist, feat32), jnp.int32),
            pltpu.VMEM((B_CHUNK, hist, FEATURES), jnp.uint16),
            pltpu.SemaphoreType.DMA,
        ],
        compiler_params=sc_params,
    )
    out = run(inputs, table_i32)
    return jax.lax.bitcast_convert_type(out, jnp.bfloat16)


# speedup vs baseline: 7.1116x; 1.0398x over previous
import functools

import jax
import jax.numpy as jnp
from jax import lax
from jax.experimental import pallas as pl
from jax.experimental.pallas import tpu as pltpu
from jax.experimental.pallas import tpu_sc as plsc

FEATURES = 32
NUM_CORES = 2
NUM_SUBCORES = 16
NUM_WORKERS = NUM_CORES * NUM_SUBCORES

B_CHUNK = 128
FIRE = 16

PACK_CHUNK = 3125


def _pack_body(n_chunks, bf16_hbm, out_hbm, buf, obuf, _sem):
    wid = lax.axis_index("s") * NUM_CORES + lax.axis_index("c")
    rows_per_w = n_chunks * PACK_CHUNK
    unroll = 5

    def chunk_body(g, carry):
        row0 = wid * rows_per_w + g * PACK_CHUNK
        pltpu.sync_copy(bf16_hbm.at[pl.ds(row0, PACK_CHUNK)], buf)

        def repack(i, c):
            for u in range(unroll):
                r = i * unroll + u
                obuf[r, :] = plsc.bitcast(buf[r, :], jnp.int32)
            return c

        lax.fori_loop(0, PACK_CHUNK // unroll, repack, 0)
        pltpu.sync_copy(obuf, out_hbm.at[pl.ds(row0, PACK_CHUNK)])
        return carry

    lax.fori_loop(0, n_chunks, chunk_body, 0)


def _embed_body(n_chunks, hist, idx_hbm, table_hbm, out_hbm, idx_v, rows_v, obuf, sem):
    wid = lax.axis_index("s") * NUM_CORES + lax.axis_index("c")
    b_per_w = n_chunks * B_CHUNK
    feat32 = FEATURES // 2

    def chunk_body(g, carry):
        b0 = wid * b_per_w + g * B_CHUNK
        pltpu.sync_copy(idx_hbm.at[pl.ds(b0, B_CHUNK)], idx_v)

        def fire_batch(ro, c):
            copies = []
            for u in range(FIRE):
                r = ro * FIRE + u
                copies.append(
                    pltpu.async_copy(
                        table_hbm.at[idx_v.at[r]],
                        rows_v.at[r],
                        sem,
                    )
                )
            for cp in copies:
                cp.wait()
            return c

        lax.fori_loop(0, B_CHUNK // FIRE, fire_batch, 0)

        def to_u16(i, c):
            for u in range(4):
                r = i * 4 + u
                b = r // hist
                h = r % hist
                obuf[b, h, :] = plsc.bitcast(rows_v[b, h, :], jnp.uint16)
            return c

        lax.fori_loop(0, B_CHUNK * hist // 4, to_u16, 0)
        pltpu.sync_copy(obuf, out_hbm.at[pl.ds(b0, B_CHUNK)])
        return carry

    lax.fori_loop(0, n_chunks, chunk_body, 0)


def kernel(inputs, embedding):
    batch, hist = inputs.shape
    assert batch % (NUM_WORKERS * B_CHUNK) == 0
    n_chunks = batch // (NUM_WORKERS * B_CHUNK)

    num_emb = embedding.shape[0]
    feat32 = FEATURES // 2
    assert num_emb % (NUM_WORKERS * PACK_CHUNK) == 0
    n_pack_chunks = num_emb // (NUM_WORKERS * PACK_CHUNK)


    mesh = plsc.VectorSubcoreMesh(core_axis_name="c", subcore_axis_name="s")
    sc_params = pltpu.CompilerParams(
        use_tc_tiling_on_sc=False, needs_layout_passes=False
    )

    pack = pl.kernel(
        functools.partial(_pack_body, n_pack_chunks),
        out_type=jax.ShapeDtypeStruct((num_emb, feat32), jnp.int32),
        mesh=mesh,
        scratch_types=[
            pltpu.VMEM((PACK_CHUNK, FEATURES), jnp.bfloat16),
            pltpu.VMEM((PACK_CHUNK, feat32), jnp.int32),
            pltpu.SemaphoreType.DMA,
        ],
        compiler_params=sc_params,
    )
    table_i32 = pack(embedding)

    run = pl.kernel(
        functools.partial(_embed_body, n_chunks, hist),
        out_type=jax.ShapeDtypeStruct((batch, hist, FEATURES), jnp.uint16),
        mesh=mesh,
        scratch_types=[
            pltpu.VMEM((B_CHUNK, hist), jnp.int32),
            pltpu.VMEM((B_CHUNK, hist, feat32), jnp.int32),
            pltpu.VMEM((B_CHUNK, hist, FEATURES), jnp.uint16),
            pltpu.SemaphoreType.DMA,
        ],
        compiler_params=sc_params,
    )
    out = run(inputs, table_i32)
    return jax.lax.bitcast_convert_type(out, jnp.bfloat16)
